# baseline (speedup 1.0000x reference)
import functools

import jax
import jax.numpy as jnp
from jax import lax
from jax.experimental import pallas as pl
from jax.experimental.pallas import tpu as pltpu
from jax.experimental.pallas import tpu_sc as plsc

_N_ENTS = 10000
_N_RELS = 200
_H = 128
_E = 320000
_B = 2048

_NC = 2
_NS = 16
_NW = _NC * _NS
_CHUNK = 128
_NCHUNK = 80
_EPW = _NCHUNK * _CHUNK
_EPAD = _EPW * _NW
_NBUF = 2

_SCH = 64
_SR = 5
_SL = 3
_K0 = 310
_K1 = 10
_TOTCH = _NS * (_K0 + _K1)
_GRP = _NCHUNK // _NBUF
_RPAD = 10240
_RPT = _RPAD // _NS

_SLOPE = (1.0 / 8.0 + 1.0 / 3.0) / 2.0


def _segsum_body(table_hbm, src_hbm, dst_hbm, zeros_hbm, out_hbm,
                 srci_v, dsti_v, rows_v, zbuf_v, acc_sh,
                 gsem, ssem, esem, dsem):
    c = lax.axis_index("c")
    s = lax.axis_index("s")
    base = jnp.where(c == 0, s * _K0, _NS * _K0 + s * _K1)
    nch = jnp.where(c == 0, _K0, _K1)
    ngrp = nch // _SR
    pltpu.sync_copy(zeros_hbm, zbuf_v)
    for z in range(_RPT // 32):
        pltpu.sync_copy(zbuf_v, acc_sh.at[pl.ds(s * _RPT + z * 32, 32)])
    plsc.subcore_barrier()

    def src_load(i, q):
        return pltpu.make_async_copy(src_hbm.at[base + i], srci_v.at[q],
                                     esem.at[q])

    def dst_load(i, q):
        return pltpu.make_async_copy(dst_hbm.at[base + i], dsti_v.at[q],
                                     dsem.at[q])

    def gath(q):
        return pltpu.make_async_copy(table_hbm.at[srci_v.at[q]],
                                     rows_v.at[q], gsem.at[q])

    def scat(q):
        return pltpu.make_async_copy(rows_v.at[q], acc_sh.at[dsti_v.at[q]],
                                     ssem.at[q])

    for q in range(_SR):
        src_load(q, q).start()
    for q in range(_SL):
        dst_load(q, q).start()
        src_load(q, q).wait()
        gath(q).start()

    def outer(g, carry):
        for b in range(_SR):
            i = g * _SR + b
            gath(b).wait()
            @pl.when(i + _SR < nch)
            def _():
                src_load(i + _SR, b).start()

            dst_load(i, b).wait()
            pltpu.async_copy(rows_v.at[b], acc_sh.at[dsti_v.at[b]],
                             ssem.at[b], add=True)

            j = i + _SL
            qq = (b + _SL) % _SR

            @pl.when(jnp.logical_and(j >= _SR, j < nch))
            def _():
                scat(qq).wait()

            @pl.when(j < nch)
            def _():
                dst_load(j, qq).start()
                src_load(j, qq).wait()
                gath(qq).start()
        return carry

    lax.fori_loop(0, ngrp, outer, 0)
    for k in range(_SR):
        scat(k).wait()
    plsc.subcore_barrier()
    pltpu.sync_copy(acc_sh.at[pl.ds(s * _RPT, _RPT)],
                    out_hbm.at[c, pl.ds(s * _RPT, _RPT)])


def _deg_body(dst_hbm, zeros_hbm, ones_hbm, deg_out,
              dstidx_v, ones_v, acc_sh, ssem):
    c = lax.axis_index("c")
    s = lax.axis_index("s")
    wid = s * _NC + c
    pltpu.sync_copy(dst_hbm.at[wid], dstidx_v)
    pltpu.sync_copy(zeros_hbm, ones_v)
    for z in range(_RPT // _CHUNK):
        pltpu.sync_copy(ones_v, acc_sh.at[pl.ds(s * _RPT + z * _CHUNK,
                                                _CHUNK)])
    pltpu.sync_copy(ones_hbm, ones_v)
    plsc.subcore_barrier()

    def outer(g, carry):
        for b in range(_NBUF):
            i = g * _NBUF + b

            @pl.when(g > 0)
            def _():
                pltpu.make_async_copy(ones_v, acc_sh.at[dstidx_v.at[i]],
                                      ssem.at[b]).wait()

            pltpu.async_copy(ones_v, acc_sh.at[dstidx_v.at[i]],
                             ssem.at[b], add=True)
        return carry

    lax.fori_loop(0, _GRP, outer, 0)
    for b in range(_NBUF):
        pltpu.make_async_copy(ones_v, acc_sh.at[dstidx_v.at[b]],
                              ssem.at[b]).wait()
    plsc.subcore_barrier()
    pltpu.sync_copy(acc_sh.at[pl.ds(s * _RPT, _RPT)],
                    deg_out.at[c, pl.ds(s * _RPT, _RPT)])


def _pairgather_body(h_hbm, rel_hbm, subj_hbm, relix_hbm, lab_hbm,
                     es_out, r_out, hlab_out, idx_v, rows_v, sem):
    c = lax.axis_index("c")
    s = lax.axis_index("s")
    wid = s * _NC + c
    base = wid * _CHUNK
    pltpu.sync_copy(subj_hbm.at[pl.ds(base, _CHUNK)], idx_v)
    pltpu.async_copy(h_hbm.at[idx_v], rows_v, sem).wait()
    pltpu.sync_copy(rows_v, es_out.at[pl.ds(base, _CHUNK)])
    pltpu.sync_copy(relix_hbm.at[pl.ds(base, _CHUNK)], idx_v)
    pltpu.async_copy(rel_hbm.at[idx_v], rows_v, sem).wait()
    pltpu.sync_copy(rows_v, r_out.at[pl.ds(base, _CHUNK)])
    pltpu.sync_copy(lab_hbm.at[pl.ds(base, _CHUNK)], idx_v)
    pltpu.async_copy(h_hbm.at[idx_v], rows_v, sem).wait()
    pltpu.sync_copy(rows_v, hlab_out.at[pl.ds(base, _CHUNK)])


@functools.cache
def _sc_kernels():
    mesh = plsc.VectorSubcoreMesh(core_axis_name="c", subcore_axis_name="s")
    segsum = pl.kernel(
        _segsum_body,
        mesh=mesh,
        out_type=jax.ShapeDtypeStruct((_NC, _RPAD, _H), jnp.float32),
        scratch_types=[
            pltpu.VMEM((_SR, _SCH), jnp.int32),
            pltpu.VMEM((_SR, _SCH), jnp.int32),
            pltpu.VMEM((_SR, _SCH, _H), jnp.float32),
            pltpu.VMEM((32, _H), jnp.float32),
            pltpu.VMEM_SHARED((_RPAD, _H), jnp.float32),
            pltpu.SemaphoreType.DMA((_SR,)),
            pltpu.SemaphoreType.DMA((_SR,)),
            pltpu.SemaphoreType.DMA((_SR,)),
            pltpu.SemaphoreType.DMA((_SR,)),
        ],
    )
    deg128 = pl.kernel(
        _deg_body,
        mesh=mesh,
        out_type=jax.ShapeDtypeStruct((_NC, _RPAD, _H), jnp.float32),
        scratch_types=[
            pltpu.VMEM((_NCHUNK, _CHUNK), jnp.int32),
            pltpu.VMEM((_CHUNK, _H), jnp.float32),
            pltpu.VMEM_SHARED((_RPAD, _H), jnp.float32),
            pltpu.SemaphoreType.DMA((_NBUF,)),
        ],
    )
    pairgather = pl.kernel(
        _pairgather_body,
        mesh=mesh,
        out_type=(jax.ShapeDtypeStruct((2 * _B, _H), jnp.float32),
                  jax.ShapeDtypeStruct((2 * _B, _H), jnp.float32),
                  jax.ShapeDtypeStruct((2 * _B, _H), jnp.float32)),
        scratch_types=[
            pltpu.VMEM((_CHUNK,), jnp.int32),
            pltpu.VMEM((_CHUNK, _H), jnp.float32),
            pltpu.SemaphoreType.DMA,
        ],
    )
    return segsum, deg128, pairgather


def _layer_body(hsum_ref, relsum_ref, deg_ref, h_ref, wnb_ref, wself_ref,
                o_ref):
    hs = (hsum_ref[0] + hsum_ref[1]) - (relsum_ref[0] + relsum_ref[1])
    deg = deg_ref[0, :, 0:1] + deg_ref[1, :, 0:1]
    norm = 1.0 / jnp.maximum(deg, 1.0)
    acc = (jnp.dot(hs * norm, wnb_ref[...], preferred_element_type=jnp.float32)
           + jnp.dot(h_ref[...], wself_ref[...],
                     preferred_element_type=jnp.float32))
    o_ref[...] = jnp.where(acc >= 0, acc, _SLOPE * acc)


_LBLK = 1280

_layer_specs_in = [
    pl.BlockSpec((_NC, _LBLK, _H), lambda i: (0, i, 0)),
    pl.BlockSpec((_NC, _LBLK, _H), lambda i: (0, i, 0)),
    pl.BlockSpec((_NC, _LBLK, _H), lambda i: (0, i, 0)),
    pl.BlockSpec((_LBLK, _H), lambda i: (i, 0)),
    pl.BlockSpec((_H, _H), lambda i: (0, 0)),
    pl.BlockSpec((_H, _H), lambda i: (0, 0)),
]
_layer_specs_out = pl.BlockSpec((_LBLK, _H), lambda i: (i, 0))

_layer = pl.pallas_call(
    _layer_body,
    grid=(_RPAD // _LBLK,),
    in_specs=_layer_specs_in,
    out_specs=_layer_specs_out,
    out_shape=jax.ShapeDtypeStruct((_RPAD, _H), jnp.float32),
)


_RB = 512
_EB = 1280


def _decoder_body(es_ref, r_ref, hlab_ref, h2_ref, wdec_ref, bdec_ref, o_ref):
    i = pl.program_id(0)
    es = es_ref[...]
    r = r_ref[...]
    q = jnp.tanh(es + r)
    x = (jnp.dot(es, wdec_ref[0:_H, :], preferred_element_type=jnp.float32)
         + jnp.dot(r, wdec_ref[_H:2 * _H, :], preferred_element_type=jnp.float32)
         + jnp.dot(q, wdec_ref[2 * _H:3 * _H, :], preferred_element_type=jnp.float32)
         + bdec_ref[...])
    x = jnp.maximum(x, 0.0)
    p = jnp.sum(x * hlab_ref[...], axis=1, keepdims=True)

    def score_tile(j):
        h2b = h2_ref[pl.ds(j * _EB, _EB), :]
        return lax.dot_general(x, h2b, (((1,), (1,)), ((), ())),
                               preferred_element_type=jnp.float32)

    def body(j, carry):
        m, sa = carry
        t = score_tile(j)
        mnew = jnp.maximum(m, jnp.max(t, axis=1, keepdims=True))
        sa = (sa * jnp.exp(m - mnew)
              + jnp.sum(jnp.exp(t - mnew), axis=1, keepdims=True))
        return (mnew, sa)

    m0 = jnp.full((_RB, 1), -jnp.inf, dtype=jnp.float32)
    s0 = jnp.zeros((_RB, 1), dtype=jnp.float32)
    nblk = _RPAD // _EB
    m, sa = lax.fori_loop(0, nblk - 1, body, (m0, s0))
    t = score_tile(nblk - 1)
    colid = ((nblk - 1) * _EB
             + lax.broadcasted_iota(jnp.int32, (_RB, _EB), 1))
    t = jnp.where(colid < _N_ENTS, t, -jnp.inf)
    mnew = jnp.maximum(m, jnp.max(t, axis=1, keepdims=True))
    sa = (sa * jnp.exp(m - mnew)
          + jnp.sum(jnp.exp(t - mnew), axis=1, keepdims=True))
    logz = jnp.log(sa) + mnew
    blocksum = (jnp.sum(logz - p) / (2.0 * _B)).reshape(1, 1)

    @pl.when(i == 0)
    def _():
        o_ref[...] = jnp.zeros((1, 1), jnp.float32)

    o_ref[...] += blocksum


_dec_specs_in = [
    pl.BlockSpec((_RB, _H), lambda i: (i, 0)),
    pl.BlockSpec((_RB, _H), lambda i: (i, 0)),
    pl.BlockSpec((_RB, _H), lambda i: (i, 0)),
    pl.BlockSpec((_RPAD, _H), lambda i: (0, 0)),
    pl.BlockSpec((3 * _H, _H), lambda i: (0, 0)),
    pl.BlockSpec((1, _H), lambda i: (0, 0)),
]
_dec_specs_out = pl.BlockSpec((1, 1), lambda i: (0, 0))

_decoder = pl.pallas_call(
    _decoder_body,
    grid=(2 * _B // _RB,),
    in_specs=_dec_specs_in,
    out_specs=_dec_specs_out,
    out_shape=jax.ShapeDtypeStruct((1, 1), jnp.float32),
)


def kernel(emb_ent, emb_rel, W1_nb, W1_self, W2_nb, W2_self, W_dec, b_dec,
           edge_index, edge_type, triples, label):
    npad = _EPAD - _E
    src_p = jnp.concatenate([edge_index[0].astype(jnp.int32),
                             jnp.zeros((npad,), jnp.int32)])
    dst_p = jnp.concatenate([edge_index[1].astype(jnp.int32),
                             jnp.full((npad,), _N_ENTS, jnp.int32)])
    et_p = jnp.concatenate([edge_type.astype(jnp.int32),
                            jnp.zeros((npad,), jnp.int32)])
    src4 = src_p.reshape(_TOTCH, _SCH)
    et4 = et_p.reshape(_TOTCH, _SCH)
    dst4 = dst_p.reshape(_TOTCH, _SCH)
    dst4d = dst_p.reshape(_NW, _NCHUNK, _CHUNK)
    zeros32 = jnp.zeros((32, _H), jnp.float32)
    zerosC = jnp.zeros((_CHUNK, _H), jnp.float32)
    ones128 = jnp.ones((_CHUNK, _H), jnp.float32)
    emb_pad = jnp.concatenate(
        [emb_ent, jnp.zeros((_RPAD - _N_ENTS, _H), jnp.float32)])

    segsum, deg128, pairgather = _sc_kernels()
    relsum = segsum(emb_rel, et4, dst4, zeros32)
    deg = deg128(dst4d, zerosC, ones128)
    hsum1 = segsum(emb_pad, src4, dst4, zeros32)
    h1 = _layer(hsum1, relsum, deg, emb_pad, W1_nb, W1_self)
    hsum2 = segsum(h1, src4, dst4, zeros32)
    h2 = _layer(hsum2, relsum, deg, h1, W2_nb, W2_self)

    subj = jnp.concatenate([triples[:, 0], triples[:, 2]]).astype(jnp.int32)
    relix = jnp.concatenate([triples[:, 1],
                             triples[:, 1] + _N_RELS]).astype(jnp.int32)
    e_s, r, hlab = pairgather(h2, emb_rel, subj, relix,
                              label.astype(jnp.int32))

    out = _decoder(e_s, r, hlab, h2, W_dec, b_dec.reshape(1, _H))
    return out[0, 0]


# speedup vs baseline: 1.0041x; 1.0041x over previous
import functools

import jax
import jax.numpy as jnp
from jax import lax
from jax.experimental import pallas as pl
from jax.experimental.pallas import tpu as pltpu
from jax.experimental.pallas import tpu_sc as plsc

_N_ENTS = 10000
_N_RELS = 200
_H = 128
_E = 320000
_B = 2048

_NC = 2
_NS = 16
_NW = _NC * _NS
_CHUNK = 128
_NCHUNK = 80
_EPW = _NCHUNK * _CHUNK
_EPAD = _EPW * _NW
_NBUF = 2

_SCH = 64
_SR = 5
_SL = 3
_K0 = 300
_K1 = 20
_TOTCH = _NS * (_K0 + _K1)
_GRP = _NCHUNK // _NBUF
_RPAD = 10240
_RPT = _RPAD // _NS

_SLOPE = (1.0 / 8.0 + 1.0 / 3.0) / 2.0


def _segsum_body(table_hbm, src_hbm, dst_hbm, zeros_hbm, out_hbm,
                 srci_v, dsti_v, rows_v, zbuf_v, acc_sh,
                 gsem, ssem, esem, dsem):
    c = lax.axis_index("c")
    s = lax.axis_index("s")
    base = jnp.where(c == 0, s * _K0, _NS * _K0 + s * _K1)
    nch = jnp.where(c == 0, _K0, _K1)
    ngrp = nch // _SR
    pltpu.sync_copy(zeros_hbm, zbuf_v)
    for z in range(_RPT // 32):
        pltpu.sync_copy(zbuf_v, acc_sh.at[pl.ds(s * _RPT + z * 32, 32)])
    plsc.subcore_barrier()

    def src_load(i, q):
        return pltpu.make_async_copy(src_hbm.at[base + i], srci_v.at[q],
                                     esem.at[q])

    def dst_load(i, q):
        return pltpu.make_async_copy(dst_hbm.at[base + i], dsti_v.at[q],
                                     dsem.at[q])

    def gath(q):
        return pltpu.make_async_copy(table_hbm.at[srci_v.at[q]],
                                     rows_v.at[q], gsem.at[q])

    def scat(q):
        return pltpu.make_async_copy(rows_v.at[q], acc_sh.at[dsti_v.at[q]],
                                     ssem.at[q])

    for q in range(_SR):
        src_load(q, q).start()
    for q in range(_SL):
        dst_load(q, q).start()
        src_load(q, q).wait()
        gath(q).start()

    def outer(g, carry):
        for b in range(_SR):
            i = g * _SR + b
            gath(b).wait()
            @pl.when(i + _SR < nch)
            def _():
                src_load(i + _SR, b).start()

            dst_load(i, b).wait()
            pltpu.async_copy(rows_v.at[b], acc_sh.at[dsti_v.at[b]],
                             ssem.at[b], add=True)

            j = i + _SL
            qq = (b + _SL) % _SR

            @pl.when(jnp.logical_and(j >= _SR, j < nch))
            def _():
                scat(qq).wait()

            @pl.when(j < nch)
            def _():
                dst_load(j, qq).start()
                src_load(j, qq).wait()
                gath(qq).start()
        return carry

    lax.fori_loop(0, ngrp, outer, 0)
    for k in range(_SR):
        scat(k).wait()
    plsc.subcore_barrier()
    pltpu.sync_copy(acc_sh.at[pl.ds(s * _RPT, _RPT)],
                    out_hbm.at[c, pl.ds(s * _RPT, _RPT)])


def _deg_body(dst_hbm, zeros_hbm, ones_hbm, deg_out,
              dstidx_v, ones_v, acc_sh, ssem):
    c = lax.axis_index("c")
    s = lax.axis_index("s")
    wid = s * _NC + c
    pltpu.sync_copy(dst_hbm.at[wid], dstidx_v)
    pltpu.sync_copy(zeros_hbm, ones_v)
    for z in range(_RPT // _CHUNK):
        pltpu.sync_copy(ones_v, acc_sh.at[pl.ds(s * _RPT + z * _CHUNK,
                                                _CHUNK)])
    pltpu.sync_copy(ones_hbm, ones_v)
    plsc.subcore_barrier()

    def outer(g, carry):
        for b in range(_NBUF):
            i = g * _NBUF + b

            @pl.when(g > 0)
            def _():
                pltpu.make_async_copy(ones_v, acc_sh.at[dstidx_v.at[i]],
                                      ssem.at[b]).wait()

            pltpu.async_copy(ones_v, acc_sh.at[dstidx_v.at[i]],
                             ssem.at[b], add=True)
        return carry

    lax.fori_loop(0, _GRP, outer, 0)
    for b in range(_NBUF):
        pltpu.make_async_copy(ones_v, acc_sh.at[dstidx_v.at[b]],
                              ssem.at[b]).wait()
    plsc.subcore_barrier()
    pltpu.sync_copy(acc_sh.at[pl.ds(s * _RPT, _RPT)],
                    deg_out.at[c, pl.ds(s * _RPT, _RPT)])


def _pairgather_body(h_hbm, rel_hbm, subj_hbm, relix_hbm, lab_hbm,
                     es_out, r_out, hlab_out, idx_v, rows_v, sem):
    c = lax.axis_index("c")
    s = lax.axis_index("s")
    wid = s * _NC + c
    base = wid * _CHUNK
    pltpu.sync_copy(subj_hbm.at[pl.ds(base, _CHUNK)], idx_v)
    pltpu.async_copy(h_hbm.at[idx_v], rows_v, sem).wait()
    pltpu.sync_copy(rows_v, es_out.at[pl.ds(base, _CHUNK)])
    pltpu.sync_copy(relix_hbm.at[pl.ds(base, _CHUNK)], idx_v)
    pltpu.async_copy(rel_hbm.at[idx_v], rows_v, sem).wait()
    pltpu.sync_copy(rows_v, r_out.at[pl.ds(base, _CHUNK)])
    pltpu.sync_copy(lab_hbm.at[pl.ds(base, _CHUNK)], idx_v)
    pltpu.async_copy(h_hbm.at[idx_v], rows_v, sem).wait()
    pltpu.sync_copy(rows_v, hlab_out.at[pl.ds(base, _CHUNK)])


@functools.cache
def _sc_kernels():
    mesh = plsc.VectorSubcoreMesh(core_axis_name="c", subcore_axis_name="s")
    segsum = pl.kernel(
        _segsum_body,
        mesh=mesh,
        out_type=jax.ShapeDtypeStruct((_NC, _RPAD, _H), jnp.float32),
        scratch_types=[
            pltpu.VMEM((_SR, _SCH), jnp.int32),
            pltpu.VMEM((_SR, _SCH), jnp.int32),
            pltpu.VMEM((_SR, _SCH, _H), jnp.float32),
            pltpu.VMEM((32, _H), jnp.float32),
            pltpu.VMEM_SHARED((_RPAD, _H), jnp.float32),
            pltpu.SemaphoreType.DMA((_SR,)),
            pltpu.SemaphoreType.DMA((_SR,)),
            pltpu.SemaphoreType.DMA((_SR,)),
            pltpu.SemaphoreType.DMA((_SR,)),
        ],
    )
    deg128 = pl.kernel(
        _deg_body,
        mesh=mesh,
        out_type=jax.ShapeDtypeStruct((_NC, _RPAD, _H), jnp.float32),
        scratch_types=[
            pltpu.VMEM((_NCHUNK, _CHUNK), jnp.int32),
            pltpu.VMEM((_CHUNK, _H), jnp.float32),
            pltpu.VMEM_SHARED((_RPAD, _H), jnp.float32),
            pltpu.SemaphoreType.DMA((_NBUF,)),
        ],
    )
    pairgather = pl.kernel(
        _pairgather_body,
        mesh=mesh,
        out_type=(jax.ShapeDtypeStruct((2 * _B, _H), jnp.float32),
                  jax.ShapeDtypeStruct((2 * _B, _H), jnp.float32),
                  jax.ShapeDtypeStruct((2 * _B, _H), jnp.float32)),
        scratch_types=[
            pltpu.VMEM((_CHUNK,), jnp.int32),
            pltpu.VMEM((_CHUNK, _H), jnp.float32),
            pltpu.SemaphoreType.DMA,
        ],
    )
    return segsum, deg128, pairgather


def _layer_body(hsum_ref, relsum_ref, deg_ref, h_ref, wnb_ref, wself_ref,
                o_ref):
    hs = (hsum_ref[0] + hsum_ref[1]) - (relsum_ref[0] + relsum_ref[1])
    deg = deg_ref[0, :, 0:1] + deg_ref[1, :, 0:1]
    norm = 1.0 / jnp.maximum(deg, 1.0)
    acc = (jnp.dot(hs * norm, wnb_ref[...], preferred_element_type=jnp.float32)
           + jnp.dot(h_ref[...], wself_ref[...],
                     preferred_element_type=jnp.float32))
    o_ref[...] = jnp.where(acc >= 0, acc, _SLOPE * acc)


_LBLK = 1280

_layer_specs_in = [
    pl.BlockSpec((_NC, _LBLK, _H), lambda i: (0, i, 0)),
    pl.BlockSpec((_NC, _LBLK, _H), lambda i: (0, i, 0)),
    pl.BlockSpec((_NC, _LBLK, _H), lambda i: (0, i, 0)),
    pl.BlockSpec((_LBLK, _H), lambda i: (i, 0)),
    pl.BlockSpec((_H, _H), lambda i: (0, 0)),
    pl.BlockSpec((_H, _H), lambda i: (0, 0)),
]
_layer_specs_out = pl.BlockSpec((_LBLK, _H), lambda i: (i, 0))

_layer = pl.pallas_call(
    _layer_body,
    grid=(_RPAD // _LBLK,),
    in_specs=_layer_specs_in,
    out_specs=_layer_specs_out,
    out_shape=jax.ShapeDtypeStruct((_RPAD, _H), jnp.float32),
)


_RB = 512
_EB = 1280


def _decoder_body(es_ref, r_ref, hlab_ref, h2_ref, wdec_ref, bdec_ref, o_ref):
    i = pl.program_id(0)
    es = es_ref[...]
    r = r_ref[...]
    q = jnp.tanh(es + r)
    x = (jnp.dot(es, wdec_ref[0:_H, :], preferred_element_type=jnp.float32)
         + jnp.dot(r, wdec_ref[_H:2 * _H, :], preferred_element_type=jnp.float32)
         + jnp.dot(q, wdec_ref[2 * _H:3 * _H, :], preferred_element_type=jnp.float32)
         + bdec_ref[...])
    x = jnp.maximum(x, 0.0)
    p = jnp.sum(x * hlab_ref[...], axis=1, keepdims=True)

    def score_tile(j):
        h2b = h2_ref[pl.ds(j * _EB, _EB), :]
        return lax.dot_general(x, h2b, (((1,), (1,)), ((), ())),
                               preferred_element_type=jnp.float32)

    def body(j, carry):
        m, sa = carry
        t = score_tile(j)
        mnew = jnp.maximum(m, jnp.max(t, axis=1, keepdims=True))
        sa = (sa * jnp.exp(m - mnew)
              + jnp.sum(jnp.exp(t - mnew), axis=1, keepdims=True))
        return (mnew, sa)

    m0 = jnp.full((_RB, 1), -jnp.inf, dtype=jnp.float32)
    s0 = jnp.zeros((_RB, 1), dtype=jnp.float32)
    nblk = _RPAD // _EB
    m, sa = lax.fori_loop(0, nblk - 1, body, (m0, s0))
    t = score_tile(nblk - 1)
    colid = ((nblk - 1) * _EB
             + lax.broadcasted_iota(jnp.int32, (_RB, _EB), 1))
    t = jnp.where(colid < _N_ENTS, t, -jnp.inf)
    mnew = jnp.maximum(m, jnp.max(t, axis=1, keepdims=True))
    sa = (sa * jnp.exp(m - mnew)
          + jnp.sum(jnp.exp(t - mnew), axis=1, keepdims=True))
    logz = jnp.log(sa) + mnew
    blocksum = (jnp.sum(logz - p) / (2.0 * _B)).reshape(1, 1)

    @pl.when(i == 0)
    def _():
        o_ref[...] = jnp.zeros((1, 1), jnp.float32)

    o_ref[...] += blocksum


_dec_specs_in = [
    pl.BlockSpec((_RB, _H), lambda i: (i, 0)),
    pl.BlockSpec((_RB, _H), lambda i: (i, 0)),
    pl.BlockSpec((_RB, _H), lambda i: (i, 0)),
    pl.BlockSpec((_RPAD, _H), lambda i: (0, 0)),
    pl.BlockSpec((3 * _H, _H), lambda i: (0, 0)),
    pl.BlockSpec((1, _H), lambda i: (0, 0)),
]
_dec_specs_out = pl.BlockSpec((1, 1), lambda i: (0, 0))

_decoder = pl.pallas_call(
    _decoder_body,
    grid=(2 * _B // _RB,),
    in_specs=_dec_specs_in,
    out_specs=_dec_specs_out,
    out_shape=jax.ShapeDtypeStruct((1, 1), jnp.float32),
)


def kernel(emb_ent, emb_rel, W1_nb, W1_self, W2_nb, W2_self, W_dec, b_dec,
           edge_index, edge_type, triples, label):
    npad = _EPAD - _E
    src_p = jnp.concatenate([edge_index[0].astype(jnp.int32),
                             jnp.zeros((npad,), jnp.int32)])
    dst_p = jnp.concatenate([edge_index[1].astype(jnp.int32),
                             jnp.full((npad,), _N_ENTS, jnp.int32)])
    et_p = jnp.concatenate([edge_type.astype(jnp.int32),
                            jnp.zeros((npad,), jnp.int32)])
    src4 = src_p.reshape(_TOTCH, _SCH)
    et4 = et_p.reshape(_TOTCH, _SCH)
    dst4 = dst_p.reshape(_TOTCH, _SCH)
    dst4d = dst_p.reshape(_NW, _NCHUNK, _CHUNK)
    zeros32 = jnp.zeros((32, _H), jnp.float32)
    zerosC = jnp.zeros((_CHUNK, _H), jnp.float32)
    ones128 = jnp.ones((_CHUNK, _H), jnp.float32)
    emb_pad = jnp.concatenate(
        [emb_ent, jnp.zeros((_RPAD - _N_ENTS, _H), jnp.float32)])

    segsum, deg128, pairgather = _sc_kernels()
    relsum = segsum(emb_rel, et4, dst4, zeros32)
    deg = deg128(dst4d, zerosC, ones128)
    hsum1 = segsum(emb_pad, src4, dst4, zeros32)
    h1 = _layer(hsum1, relsum, deg, emb_pad, W1_nb, W1_self)
    hsum2 = segsum(h1, src4, dst4, zeros32)
    h2 = _layer(hsum2, relsum, deg, h1, W2_nb, W2_self)

    subj = jnp.concatenate([triples[:, 0], triples[:, 2]]).astype(jnp.int32)
    relix = jnp.concatenate([triples[:, 1],
                             triples[:, 1] + _N_RELS]).astype(jnp.int32)
    e_s, r, hlab = pairgather(h2, emb_rel, subj, relix,
                              label.astype(jnp.int32))

    out = _decoder(e_s, r, hlab, h2, W_dec, b_dec.reshape(1, _H))
    return out[0, 0]
